# baseline (speedup 1.0000x reference)
import functools
import math

import jax
import jax.numpy as jnp
from jax import lax
from jax.experimental import pallas as pl
from jax.experimental.pallas import tpu as pltpu
from jax.experimental.pallas import tpu_sc as plsc

D_MODEL_K = 64
VOCAB_K = 1_000_000
SCALE = math.sqrt(D_MODEL_K)

NC = 2
NS = 16
NW = NC * NS

SEQ = 200
BATCH = 4096
B_TOTAL = BATCH * SEQ
B_PER_W = B_TOTAL // NW
CHUNK = 256
BLOCKS_PER_S = BATCH // CHUNK
NCHUNK = B_PER_W // CHUNK


@functools.partial(
    pl.kernel,
    out_type=jax.ShapeDtypeStruct((SEQ, D_MODEL_K, BATCH), jnp.float32),
    mesh=plsc.VectorSubcoreMesh(core_axis_name="c", subcore_axis_name="s"),
    compiler_params=pltpu.CompilerParams(
        use_tc_tiling_on_sc=False, needs_layout_passes=False
    ),
    scratch_types=[
        pltpu.VMEM((B_PER_W,), jnp.int32),
        pltpu.VMEM((CHUNK, D_MODEL_K), jnp.float32),
        pltpu.VMEM((CHUNK, D_MODEL_K), jnp.float32),
        pltpu.VMEM((D_MODEL_K, CHUNK), jnp.float32),
        pltpu.VMEM((D_MODEL_K, CHUNK), jnp.float32),
        pltpu.SemaphoreType.DMA((2,)),
        pltpu.SemaphoreType.DMA((2,)),
    ],
)
def _emb_lookup(table_hbm, x_hbm, out_hbm, idx_v, rows_a, rows_b,
                out_a, out_b, gsem, osem):
    wid = lax.axis_index("s") * NC + lax.axis_index("c")
    base_tok = wid * B_PER_W
    base_c = wid * NCHUNK
    rows_bufs = (rows_a, rows_b)
    out_bufs = (out_a, out_b)

    pltpu.sync_copy(x_hbm.at[pl.ds(base_tok, B_PER_W)], idx_v)

    def issue_gather(i, X):
        pltpu.async_copy(
            table_hbm.at[idx_v.at[pl.ds(i * CHUNK, CHUNK)]],
            rows_bufs[X],
            gsem.at[X],
        )

    def wait_gather(X):
        pltpu.make_async_copy(
            table_hbm.at[pl.ds(0, CHUNK)], rows_bufs[X], gsem.at[X]
        ).wait()

    def issue_out(i, X):
        c = base_c + i
        s = c // BLOCKS_PER_S
        b0 = (c % BLOCKS_PER_S) * CHUNK
        pltpu.async_copy(
            out_bufs[X], out_hbm.at[s, :, pl.ds(b0, CHUNK)], osem.at[X]
        )

    def wait_out(X):
        pltpu.make_async_copy(
            out_bufs[X], out_hbm.at[0, :, pl.ds(0, CHUNK)], osem.at[X]
        ).wait()

    def transpose_scale(X):
        rows, outv = rows_bufs[X], out_bufs[X]

        @plsc.parallel_loop(0, CHUNK // 16, 1)
        def _(tb):
            row_ids = tb * 16 + lax.iota(jnp.int32, 16)
            for d in range(D_MODEL_K):
                col_ids = jnp.full((16,), d, jnp.int32)
                vals = plsc.load_gather(rows, [row_ids, col_ids])
                outv[d, pl.ds(tb * 16, 16)] = vals * SCALE

    issue_gather(0, 0)

    def body(j, carry):
        i = 2 * j
        issue_gather(i + 1, 1)
        wait_gather(0)

        @pl.when(j >= 1)
        def _():
            wait_out(0)

        transpose_scale(0)
        issue_out(i, 0)

        @pl.when(j < NCHUNK // 2 - 1)
        def _():
            issue_gather(i + 2, 0)

        wait_gather(1)

        @pl.when(j >= 1)
        def _():
            wait_out(1)

        transpose_scale(1)
        issue_out(i + 1, 1)
        return carry

    lax.fori_loop(0, NCHUNK // 2, body, 0)
    wait_out(0)
    wait_out(1)


def kernel(x, table):
    xf = x.T.reshape(-1).astype(jnp.int32)
    out_t = _emb_lookup(table, xf)
    return out_t.transpose(2, 0, 1)


# speedup vs baseline: 1.6027x; 1.6027x over previous
import functools
import math

import jax
import jax.numpy as jnp
from jax import lax
from jax.experimental import pallas as pl
from jax.experimental.pallas import tpu as pltpu
from jax.experimental.pallas import tpu_sc as plsc

D_MODEL_K = 64
VOCAB_K = 1_000_000
SCALE = math.sqrt(D_MODEL_K)

NC = 2
NS = 16
NW = NC * NS

SEQ = 200
BATCH = 4096
B_TOTAL = BATCH * SEQ
B_PER_W = B_TOTAL // NW
CHUNK = 256
PITCH = CHUNK + 1
BLOCKS_PER_S = BATCH // CHUNK
NCHUNK = B_PER_W // CHUNK


@functools.partial(
    pl.kernel,
    out_type=jax.ShapeDtypeStruct((SEQ, D_MODEL_K, BATCH), jnp.float32),
    mesh=plsc.VectorSubcoreMesh(core_axis_name="c", subcore_axis_name="s"),
    compiler_params=pltpu.CompilerParams(
        use_tc_tiling_on_sc=False, needs_layout_passes=False
    ),
    scratch_types=[
        pltpu.VMEM((B_PER_W,), jnp.int32),
        pltpu.VMEM((CHUNK, D_MODEL_K), jnp.float32),
        pltpu.VMEM((CHUNK, D_MODEL_K), jnp.float32),
        pltpu.VMEM((D_MODEL_K, PITCH), jnp.float32),
        pltpu.VMEM((D_MODEL_K, PITCH), jnp.float32),
        pltpu.SemaphoreType.DMA((2,)),
        pltpu.SemaphoreType.DMA((2,)),
    ],
)
def _emb_lookup(table_hbm, x_hbm, out_hbm, idx_v, rows_a, rows_b,
                out_a, out_b, gsem, osem):
    wid = lax.axis_index("s") * NC + lax.axis_index("c")
    base_tok = wid * B_PER_W
    base_c = wid * NCHUNK
    rows_bufs = (rows_a, rows_b)
    out_bufs = (out_a, out_b)

    pltpu.sync_copy(x_hbm.at[pl.ds(base_tok, B_PER_W)], idx_v)

    def issue_gather(i, X):
        pltpu.async_copy(
            table_hbm.at[idx_v.at[pl.ds(i * CHUNK, CHUNK)]],
            rows_bufs[X],
            gsem.at[X],
        )

    def wait_gather(X):
        pltpu.make_async_copy(
            table_hbm.at[pl.ds(0, CHUNK)], rows_bufs[X], gsem.at[X]
        ).wait()

    def issue_out(i, X):
        c = base_c + i
        s = c // BLOCKS_PER_S
        b0 = (c % BLOCKS_PER_S) * CHUNK
        pltpu.async_copy(
            out_bufs[X].at[:, pl.ds(0, CHUNK)],
            out_hbm.at[s, :, pl.ds(b0, CHUNK)],
            osem.at[X],
        )

    def wait_out(X):
        pltpu.make_async_copy(
            out_bufs[X].at[:, pl.ds(0, CHUNK)],
            out_hbm.at[0, :, pl.ds(0, CHUNK)],
            osem.at[X],
        ).wait()

    def transpose_scale(X):
        rows, outv = rows_bufs[X], out_bufs[X]
        lane = lax.iota(jnp.int32, 16)

        @plsc.parallel_loop(0, CHUNK, 1, unroll=4)
        def _(t):
            col = jnp.full((16,), t, jnp.int32)
            for d0 in range(0, D_MODEL_K, 16):
                vals = rows[t, pl.ds(d0, 16)] * SCALE
                plsc.store_scatter(outv, [d0 + lane, col], vals)

    issue_gather(0, 0)

    def body(j, carry):
        i = 2 * j
        issue_gather(i + 1, 1)
        wait_gather(0)

        @pl.when(j >= 1)
        def _():
            wait_out(0)

        transpose_scale(0)
        issue_out(i, 0)

        @pl.when(j < NCHUNK // 2 - 1)
        def _():
            issue_gather(i + 2, 0)

        wait_gather(1)

        @pl.when(j >= 1)
        def _():
            wait_out(1)

        transpose_scale(1)
        issue_out(i + 1, 1)
        return carry

    lax.fori_loop(0, NCHUNK // 2, body, 0)
    wait_out(0)
    wait_out(1)


def kernel(x, table):
    xf = x.T.reshape(-1).astype(jnp.int32)
    out_t = _emb_lookup(table, xf)
    return out_t.transpose(2, 0, 1)


# speedup vs baseline: 2.0332x; 1.2686x over previous
import functools
import math

import jax
import jax.numpy as jnp
from jax import lax
from jax.experimental import pallas as pl
from jax.experimental.pallas import tpu as pltpu
from jax.experimental.pallas import tpu_sc as plsc

D_MODEL_K = 64
VOCAB_K = 1_000_000
SCALE = math.sqrt(D_MODEL_K)

NC = 2
NS = 16
NW = NC * NS

SEQ = 200
BATCH = 4096
B_TOTAL = BATCH * SEQ
B_PER_W = B_TOTAL // NW
CHUNK = 256
PITCH = CHUNK + 1
BLOCKS_PER_S = BATCH // CHUNK
NCHUNK = B_PER_W // CHUNK


@functools.partial(
    pl.kernel,
    out_type=jax.ShapeDtypeStruct((SEQ, 8, BATCH // 128, 8, 128), jnp.float32),
    mesh=plsc.VectorSubcoreMesh(core_axis_name="c", subcore_axis_name="s"),
    compiler_params=pltpu.CompilerParams(
        use_tc_tiling_on_sc=False, needs_layout_passes=False
    ),
    scratch_types=[
        pltpu.VMEM((B_PER_W,), jnp.int32),
        pltpu.VMEM((CHUNK, D_MODEL_K), jnp.float32),
        pltpu.VMEM((CHUNK, D_MODEL_K), jnp.float32),
        pltpu.VMEM((D_MODEL_K, PITCH), jnp.float32),
        pltpu.VMEM((D_MODEL_K, PITCH), jnp.float32),
        pltpu.SemaphoreType.DMA((2,)),
        pltpu.SemaphoreType.DMA((2,)),
    ],
)
def _emb_lookup(table_hbm, x_hbm, out_hbm, idx_v, rows_a, rows_b,
                out_a, out_b, gsem, osem):
    wid = lax.axis_index("s") * NC + lax.axis_index("c")
    base_tok = wid * B_PER_W
    base_c = wid * NCHUNK
    rows_bufs = (rows_a, rows_b)
    out_bufs = (out_a, out_b)

    pltpu.sync_copy(x_hbm.at[pl.ds(base_tok, B_PER_W)], idx_v)

    def issue_gather(i, X):
        pltpu.async_copy(
            table_hbm.at[idx_v.at[pl.ds(i * CHUNK, CHUNK)]],
            rows_bufs[X],
            gsem.at[X],
        )

    def wait_gather(X):
        pltpu.make_async_copy(
            table_hbm.at[pl.ds(0, CHUNK)], rows_bufs[X], gsem.at[X]
        ).wait()

    def issue_out(i, X):
        c = base_c + i
        s = c // BLOCKS_PER_S
        bc0 = (c % BLOCKS_PER_S) * (CHUNK // 128)
        for dr in range(D_MODEL_K // 8):
            for bcg in range(CHUNK // 128):
                pltpu.async_copy(
                    out_bufs[X].at[pl.ds(dr * 8, 8), pl.ds(bcg * 128, 128)],
                    out_hbm.at[s, dr, bc0 + bcg, :, :],
                    osem.at[X],
                )

    def wait_out(X):
        for _ in range(D_MODEL_K // 8 * (CHUNK // 128)):
            pltpu.make_async_copy(
                out_bufs[X].at[pl.ds(0, 8), pl.ds(0, 128)],
                out_hbm.at[0, 0, 0, :, :],
                osem.at[X],
            ).wait()

    def transpose_scale(X):
        rows, outv = rows_bufs[X], out_bufs[X]
        lane = lax.iota(jnp.int32, 16)

        @plsc.parallel_loop(0, CHUNK, 1, unroll=4)
        def _(t):
            col = jnp.full((16,), t, jnp.int32)
            for d0 in range(0, D_MODEL_K, 16):
                vals = rows[t, pl.ds(d0, 16)] * SCALE
                plsc.store_scatter(outv, [d0 + lane, col], vals)

    issue_gather(0, 0)

    def body(j, carry):
        i = 2 * j
        issue_gather(i + 1, 1)
        wait_gather(0)

        @pl.when(j >= 1)
        def _():
            wait_out(0)

        transpose_scale(0)
        issue_out(i, 0)

        @pl.when(j < NCHUNK // 2 - 1)
        def _():
            issue_gather(i + 2, 0)

        wait_gather(1)

        @pl.when(j >= 1)
        def _():
            wait_out(1)

        transpose_scale(1)
        issue_out(i + 1, 1)
        return carry

    lax.fori_loop(0, NCHUNK // 2, body, 0)
    wait_out(0)
    wait_out(1)


def kernel(x, table):
    xf = x.T.reshape(-1).astype(jnp.int32)
    out5 = _emb_lookup(table, xf)
    return jnp.transpose(out5, (2, 4, 0, 1, 3)).reshape(BATCH, SEQ, D_MODEL_K)


# speedup vs baseline: 2.0377x; 1.0022x over previous
import functools
import math

import jax
import jax.numpy as jnp
from jax import lax
from jax.experimental import pallas as pl
from jax.experimental.pallas import tpu as pltpu
from jax.experimental.pallas import tpu_sc as plsc

D_MODEL_K = 64
VOCAB_K = 1_000_000
SCALE = math.sqrt(D_MODEL_K)

NC = 2
NS = 16
NW = NC * NS

SEQ = 200
BATCH = 4096
B_TOTAL = BATCH * SEQ
B_PER_W = B_TOTAL // NW
CHUNK = 256
PITCH = CHUNK + 1
BLOCKS_PER_S = BATCH // CHUNK
NCHUNK = B_PER_W // CHUNK


@functools.partial(
    pl.kernel,
    out_type=jax.ShapeDtypeStruct((SEQ, 8, BATCH // 128, 8, 128), jnp.float32),
    mesh=plsc.VectorSubcoreMesh(core_axis_name="c", subcore_axis_name="s"),
    compiler_params=pltpu.CompilerParams(
        use_tc_tiling_on_sc=False, needs_layout_passes=False
    ),
    scratch_types=[
        pltpu.VMEM((B_PER_W,), jnp.int32),
        pltpu.VMEM((CHUNK, D_MODEL_K), jnp.float32),
        pltpu.VMEM((CHUNK, D_MODEL_K), jnp.float32),
        pltpu.VMEM((8, 8, PITCH), jnp.float32),
        pltpu.VMEM((8, 8, PITCH), jnp.float32),
        pltpu.SemaphoreType.DMA((2,)),
        pltpu.SemaphoreType.DMA((2,)),
    ],
)
def _emb_lookup(table_hbm, x_hbm, out_hbm, idx_v, rows_a, rows_b,
                out_a, out_b, gsem, osem):
    wid = lax.axis_index("s") * NC + lax.axis_index("c")
    base_tok = wid * B_PER_W
    base_c = wid * NCHUNK
    rows_bufs = (rows_a, rows_b)
    out_bufs = (out_a, out_b)

    pltpu.sync_copy(x_hbm.at[pl.ds(base_tok, B_PER_W)], idx_v)

    def issue_gather(i, X):
        pltpu.async_copy(
            table_hbm.at[idx_v.at[pl.ds(i * CHUNK, CHUNK)]],
            rows_bufs[X],
            gsem.at[X],
        )

    def wait_gather(X):
        pltpu.make_async_copy(
            table_hbm.at[pl.ds(0, CHUNK)], rows_bufs[X], gsem.at[X]
        ).wait()

    def issue_out(i, X):
        c = base_c + i
        s = c // BLOCKS_PER_S
        bc0 = (c % BLOCKS_PER_S) * (CHUNK // 128)
        for bcg in range(CHUNK // 128):
            pltpu.async_copy(
                out_bufs[X].at[:, :, pl.ds(bcg * 128, 128)],
                out_hbm.at[s, :, bc0 + bcg, :, :],
                osem.at[X],
            )

    def wait_out(X):
        for _ in range(CHUNK // 128):
            pltpu.make_async_copy(
                out_bufs[X].at[:, :, pl.ds(0, 128)],
                out_hbm.at[0, :, 0, :, :],
                osem.at[X],
            ).wait()

    def transpose_scale(X):
        rows, outv = rows_bufs[X], out_bufs[X]
        lane = lax.iota(jnp.int32, 16)

        @plsc.parallel_loop(0, CHUNK, 1, unroll=4)
        def _(t):
            col = jnp.full((16,), t, jnp.int32)
            for d0 in range(0, D_MODEL_K, 16):
                vals = rows[t, pl.ds(d0, 16)] * SCALE
                dr = lax.shift_right_logical(d0 + lane, 3)
                di = lax.bitwise_and(d0 + lane, jnp.int32(7))
                plsc.store_scatter(outv, [dr, di, col], vals)

    issue_gather(0, 0)

    def body(j, carry):
        i = 2 * j
        issue_gather(i + 1, 1)
        wait_gather(0)

        @pl.when(j >= 1)
        def _():
            wait_out(0)

        transpose_scale(0)
        issue_out(i, 0)

        @pl.when(j < NCHUNK // 2 - 1)
        def _():
            issue_gather(i + 2, 0)

        wait_gather(1)

        @pl.when(j >= 1)
        def _():
            wait_out(1)

        transpose_scale(1)
        issue_out(i + 1, 1)
        return carry

    lax.fori_loop(0, NCHUNK // 2, body, 0)
    wait_out(0)
    wait_out(1)


def kernel(x, table):
    xf = x.T.reshape(-1).astype(jnp.int32)
    out5 = _emb_lookup(table, xf)
    return jnp.transpose(out5, (2, 4, 0, 1, 3)).reshape(BATCH, SEQ, D_MODEL_K)
